# baseline (device time: 24298 ns/iter reference)
import jax
import jax.numpy as jnp
from jax import lax
from jax.experimental import pallas as pl
from jax.experimental.pallas import tpu as pltpu

N_DEV = 8
B, SQ, HQ, DH = 2, 128, 4, 64
DM = 512
DQ = HQ * DH
BLK = 64


def kernel(x, Wq, K_ext, V_ext, Wo):
    def body(x_ref, wq_ref, k_ref, v_ref, wo_ref, out_ref,
             ctx_ref, send_sems, recv_sem):
        my = lax.axis_index("i")

        @pl.when(my == 0)
        def _():
            xf = x_ref[...].reshape(B * SQ, DM)
            q = jnp.dot(xf, wq_ref[...], preferred_element_type=jnp.float32)
            q = q.reshape(B, SQ, HQ, DH)
            k = k_ref[...]
            v = v_ref[...]
            row = lax.broadcasted_iota(jnp.int32, (SQ, SQ), 0) // BLK
            col = lax.broadcasted_iota(jnp.int32, (SQ, SQ), 1) // BLK
            mask = col <= row
            for b in range(B):
                for h in range(HQ):
                    qh = q[b, :, h, :]
                    kh = k[b, :, h, :]
                    vh = v[b, :, h, :]
                    s = lax.dot_general(
                        qh, kh, (((1,), (1,)), ((), ())),
                        preferred_element_type=jnp.float32,
                    ) * 0.125
                    s = jnp.where(mask, s, -1e9)
                    m = jnp.max(s, axis=-1, keepdims=True)
                    w = jnp.exp(s - m)
                    w = w / jnp.sum(w, axis=-1, keepdims=True)
                    ctx_ref[b * SQ:(b + 1) * SQ, h * DH:(h + 1) * DH] = (
                        jnp.dot(w, vh, preferred_element_type=jnp.float32))

            sends = []
            for j in range(1, N_DEV):
                rdma = pltpu.make_async_remote_copy(
                    src_ref=ctx_ref,
                    dst_ref=ctx_ref,
                    send_sem=send_sems.at[j - 1],
                    recv_sem=recv_sem,
                    device_id=(j,),
                    device_id_type=pl.DeviceIdType.MESH,
                )
                rdma.start()
                sends.append(rdma)
            for rdma in sends:
                rdma.wait_send()

        @pl.when(my != 0)
        def _():
            rdma = pltpu.make_async_remote_copy(
                src_ref=ctx_ref,
                dst_ref=ctx_ref,
                send_sem=send_sems.at[0],
                recv_sem=recv_sem,
                device_id=(0,),
                device_id_type=pl.DeviceIdType.MESH,
            )
            rdma.wait_recv()

        out = jnp.dot(ctx_ref[...], wo_ref[...],
                      preferred_element_type=jnp.float32)
        out_ref[...] = out.reshape(B, SQ, DM)

    return pl.pallas_call(
        body,
        out_shape=jax.ShapeDtypeStruct((B, SQ, DM), jnp.float32),
        in_specs=[pl.BlockSpec(memory_space=pltpu.VMEM)] * 5,
        out_specs=pl.BlockSpec(memory_space=pltpu.VMEM),
        scratch_shapes=[
            pltpu.VMEM((B * SQ, DQ), jnp.float32),
            pltpu.SemaphoreType.DMA((N_DEV - 1,)),
            pltpu.SemaphoreType.DMA,
        ],
    )(x, Wq, K_ext, V_ext, Wo)


# device time: 15819 ns/iter; 1.5360x vs baseline; 1.5360x over previous
import jax
import jax.numpy as jnp
from jax import lax
from jax.experimental import pallas as pl
from jax.experimental.pallas import tpu as pltpu

N_DEV = 8
B, SQ, HQ, DH = 2, 128, 4, 64
DM = 512
DQ = HQ * DH
BLK = 64


def kernel(x, Wq, K_ext, V_ext, Wo):
    def body(x_ref, wq_ref, k_ref, v_ref, wo_ref, out_ref,
             ctx_ref, send_sems, recv_sem):
        my = lax.axis_index("i")

        def send_to(targets):
            sends = []
            for i, j in enumerate(targets):
                rdma = pltpu.make_async_remote_copy(
                    src_ref=ctx_ref,
                    dst_ref=ctx_ref,
                    send_sem=send_sems.at[i],
                    recv_sem=recv_sem,
                    device_id=(j,),
                    device_id_type=pl.DeviceIdType.MESH,
                )
                rdma.start()
                sends.append(rdma)
            for rdma in sends:
                rdma.wait_send()

        def wait_recv():
            rdma = pltpu.make_async_remote_copy(
                src_ref=ctx_ref,
                dst_ref=ctx_ref,
                send_sem=send_sems.at[0],
                recv_sem=recv_sem,
                device_id=(0,),
                device_id_type=pl.DeviceIdType.MESH,
            )
            rdma.wait_recv()

        @pl.when(my == 0)
        def _():
            xf = x_ref[...].reshape(B * SQ, DM)
            q = jnp.dot(xf, wq_ref[...], preferred_element_type=jnp.float32)
            q = q.reshape(B, SQ, HQ, DH)
            k = k_ref[...]
            v = v_ref[...]
            row = lax.broadcasted_iota(jnp.int32, (SQ, SQ), 0) // BLK
            col = lax.broadcasted_iota(jnp.int32, (SQ, SQ), 1) // BLK
            mask = col <= row
            for b in range(B):
                for h in range(HQ):
                    qh = q[b, :, h, :]
                    kh = k[b, :, h, :]
                    vh = v[b, :, h, :]
                    s = lax.dot_general(
                        qh, kh, (((1,), (1,)), ((), ())),
                        preferred_element_type=jnp.float32,
                    ) * 0.125
                    s = jnp.where(mask, s, -1e9)
                    m = jnp.max(s, axis=-1, keepdims=True)
                    w = jnp.exp(s - m)
                    w = w / jnp.sum(w, axis=-1, keepdims=True)
                    ctx = jnp.dot(w, vh, preferred_element_type=jnp.float32)
                    ctx_ref[b * SQ:(b + 1) * SQ, h * DH:(h + 1) * DH] = (
                        ctx.astype(jnp.bfloat16))
            send_to([1, 2, 3, 4])

        @pl.when(my == 4)
        def _():
            wait_recv()
            send_to([5, 6, 7])

        @pl.when(jnp.logical_and(my != 0, my != 4))
        def _():
            wait_recv()

        ctxf = ctx_ref[...].astype(jnp.float32)
        out = jnp.dot(ctxf, wo_ref[...], preferred_element_type=jnp.float32)
        out_ref[...] = out.reshape(B, SQ, DM)

    return pl.pallas_call(
        body,
        out_shape=jax.ShapeDtypeStruct((B, SQ, DM), jnp.float32),
        in_specs=[pl.BlockSpec(memory_space=pltpu.VMEM)] * 5,
        out_specs=pl.BlockSpec(memory_space=pltpu.VMEM),
        scratch_shapes=[
            pltpu.VMEM((B * SQ, DQ), jnp.bfloat16),
            pltpu.SemaphoreType.DMA((4,)),
            pltpu.SemaphoreType.DMA,
        ],
    )(x, Wq, K_ext, V_ext, Wo)


# device time: 11001 ns/iter; 2.2087x vs baseline; 1.4380x over previous
import jax
import jax.numpy as jnp
from jax import lax
from jax.experimental import pallas as pl
from jax.experimental.pallas import tpu as pltpu

N_DEV = 8
B, SQ, HQ, DH = 2, 128, 4, 64
DM = 512
DQ = HQ * DH
BLK = 64


def kernel(x, Wq, K_ext, V_ext, Wo):
    def body(x_ref, wq_ref, k_ref, v_ref, wo_ref, out_ref,
             ctx_ref, send_sems, recv_sem):
        my = lax.axis_index("i")

        barrier_sem = pltpu.get_barrier_semaphore()

        @pl.when(jnp.logical_and(my >= 1, my <= 4))
        def _():
            pl.semaphore_signal(barrier_sem, inc=1, device_id=(0,),
                                device_id_type=pl.DeviceIdType.MESH)

        @pl.when(my >= 5)
        def _():
            pl.semaphore_signal(barrier_sem, inc=1, device_id=(4,),
                                device_id_type=pl.DeviceIdType.MESH)

        def send_to(targets):
            sends = []
            for i, j in enumerate(targets):
                rdma = pltpu.make_async_remote_copy(
                    src_ref=ctx_ref,
                    dst_ref=ctx_ref,
                    send_sem=send_sems.at[i],
                    recv_sem=recv_sem,
                    device_id=(j,),
                    device_id_type=pl.DeviceIdType.MESH,
                )
                rdma.start()
                sends.append(rdma)
            for rdma in sends:
                rdma.wait_send()

        def wait_recv():
            rdma = pltpu.make_async_remote_copy(
                src_ref=ctx_ref,
                dst_ref=ctx_ref,
                send_sem=send_sems.at[0],
                recv_sem=recv_sem,
                device_id=(0,),
                device_id_type=pl.DeviceIdType.MESH,
            )
            rdma.wait_recv()

        @pl.when(my == 0)
        def _():
            xf = x_ref[...].reshape(B * SQ, DM)
            q = jnp.dot(xf, wq_ref[...], preferred_element_type=jnp.float32)
            q = q.reshape(B, SQ, HQ, DH)
            k = k_ref[...]
            v = v_ref[...]
            row = lax.broadcasted_iota(jnp.int32, (SQ, SQ), 0) // BLK
            col = lax.broadcasted_iota(jnp.int32, (SQ, SQ), 1) // BLK
            mask = col <= row
            for b in range(B):
                for h in range(HQ):
                    qh = q[b, :, h, :]
                    kh = k[b, :, h, :]
                    vh = v[b, :, h, :]
                    s = lax.dot_general(
                        qh, kh, (((1,), (1,)), ((), ())),
                        preferred_element_type=jnp.float32,
                    ) * 0.125
                    s = jnp.where(mask, s, -1e9)
                    m = jnp.max(s, axis=-1, keepdims=True)
                    w = jnp.exp(s - m)
                    w = w / jnp.sum(w, axis=-1, keepdims=True)
                    ctx = jnp.dot(w, vh, preferred_element_type=jnp.float32)
                    ctx_ref[b * SQ:(b + 1) * SQ, h * DH:(h + 1) * DH] = (
                        ctx.astype(jnp.bfloat16))
            pl.semaphore_wait(barrier_sem, 4)
            send_to([1, 2, 3, 4])

        @pl.when(my == 4)
        def _():
            wait_recv()
            pl.semaphore_wait(barrier_sem, 3)
            send_to([5, 6, 7])

        @pl.when(jnp.logical_and(my != 0, my != 4))
        def _():
            wait_recv()

        ctxf = ctx_ref[...].astype(jnp.float32)
        out = jnp.dot(ctxf, wo_ref[...], preferred_element_type=jnp.float32)
        out_ref[...] = out.reshape(B, SQ, DM)

    return pl.pallas_call(
        body,
        out_shape=jax.ShapeDtypeStruct((B, SQ, DM), jnp.float32),
        in_specs=[pl.BlockSpec(memory_space=pltpu.VMEM)] * 5,
        out_specs=pl.BlockSpec(memory_space=pltpu.VMEM),
        scratch_shapes=[
            pltpu.VMEM((B * SQ, DQ), jnp.bfloat16),
            pltpu.SemaphoreType.DMA((4,)),
            pltpu.SemaphoreType.DMA,
        ],
        compiler_params=pltpu.CompilerParams(collective_id=0),
    )(x, Wq, K_ext, V_ext, Wo)


# device time: 10917 ns/iter; 2.2257x vs baseline; 1.0077x over previous
import jax
import jax.numpy as jnp
from jax import lax
from jax.experimental import pallas as pl
from jax.experimental.pallas import tpu as pltpu

N_DEV = 8
B, SQ, HQ, DH = 2, 128, 4, 64
DM = 512
DQ = HQ * DH
BLK = 64


def kernel(x, Wq, K_ext, V_ext, Wo):
    def body(x_ref, wq_ref, k_ref, v_ref, wo_ref, out_ref,
             ctx_ref, send_sems, recv_sems):
        my = lax.axis_index("i")

        barrier_sem = pltpu.get_barrier_semaphore()

        @pl.when(jnp.logical_and(my >= 1, my <= 4))
        def _():
            pl.semaphore_signal(barrier_sem, inc=1, device_id=(0,),
                                device_id_type=pl.DeviceIdType.MESH)

        @pl.when(my >= 5)
        def _():
            pl.semaphore_signal(barrier_sem, inc=1, device_id=(4,),
                                device_id_type=pl.DeviceIdType.MESH)

        def start_sends(b, targets):
            descs = []
            for i, j in enumerate(targets):
                rdma = pltpu.make_async_remote_copy(
                    src_ref=ctx_ref.at[b],
                    dst_ref=ctx_ref.at[b],
                    send_sem=send_sems.at[b, i],
                    recv_sem=recv_sems.at[b],
                    device_id=(j,),
                    device_id_type=pl.DeviceIdType.MESH,
                )
                rdma.start()
                descs.append(rdma)
            return descs

        def wait_recv(b):
            rdma = pltpu.make_async_remote_copy(
                src_ref=ctx_ref.at[b],
                dst_ref=ctx_ref.at[b],
                send_sem=send_sems.at[b, 0],
                recv_sem=recv_sems.at[b],
                device_id=(0,),
                device_id_type=pl.DeviceIdType.MESH,
            )
            rdma.wait_recv()

        def out_rows(b):
            ctxb = ctx_ref[b].astype(jnp.float32)
            out_ref[b, :, :] = jnp.dot(ctxb, wo_ref[...],
                                       preferred_element_type=jnp.float32)

        @pl.when(my == 0)
        def _():
            xf = x_ref[...].reshape(B * SQ, DM)
            q2 = jnp.dot(xf, wq_ref[...], preferred_element_type=jnp.float32)
            k2 = k_ref[...].reshape(B * SQ, DQ)
            v2 = v_ref[...].reshape(B * SQ, DQ)
            row = lax.broadcasted_iota(jnp.int32, (SQ, SQ), 0) // BLK
            col = lax.broadcasted_iota(jnp.int32, (SQ, SQ), 1) // BLK
            mask = col <= row
            sends = []
            for b in range(B):
                rows = slice(b * SQ, (b + 1) * SQ)
                for h in range(HQ):
                    cols = slice(h * DH, (h + 1) * DH)
                    qh = q2[rows, cols]
                    kh = k2[rows, cols]
                    vh = v2[rows, cols]
                    s = lax.dot_general(
                        qh, kh, (((1,), (1,)), ((), ())),
                        preferred_element_type=jnp.float32,
                    ) * 0.125
                    s = jnp.where(mask, s, -1e9)
                    m = jnp.max(s, axis=-1, keepdims=True)
                    w = jnp.exp(s - m)
                    w = w / jnp.sum(w, axis=-1, keepdims=True)
                    ctx = jnp.dot(w, vh, preferred_element_type=jnp.float32)
                    ctx_ref[b, :, cols] = ctx.astype(jnp.bfloat16)
                if b == 0:
                    pl.semaphore_wait(barrier_sem, 4)
                sends += start_sends(b, [1, 2, 3, 4])
            for b in range(B):
                out_rows(b)
            for rdma in sends:
                rdma.wait_send()

        @pl.when(my == 4)
        def _():
            sends = []
            for b in range(B):
                wait_recv(b)
                if b == 0:
                    pl.semaphore_wait(barrier_sem, 3)
                sends += start_sends(b, [5, 6, 7])
                out_rows(b)
            for rdma in sends:
                rdma.wait_send()

        @pl.when(jnp.logical_and(my != 0, my != 4))
        def _():
            for b in range(B):
                wait_recv(b)
                out_rows(b)

    return pl.pallas_call(
        body,
        out_shape=jax.ShapeDtypeStruct((B, SQ, DM), jnp.float32),
        in_specs=[pl.BlockSpec(memory_space=pltpu.VMEM)] * 5,
        out_specs=pl.BlockSpec(memory_space=pltpu.VMEM),
        scratch_shapes=[
            pltpu.VMEM((B, SQ, DQ), jnp.bfloat16),
            pltpu.SemaphoreType.DMA((B, 4)),
            pltpu.SemaphoreType.DMA((B,)),
        ],
        compiler_params=pltpu.CompilerParams(collective_id=0),
    )(x, Wq, K_ext, V_ext, Wo)
